# baseline (speedup 1.0000x reference)
import functools

import jax
import jax.numpy as jnp
from jax import lax
from jax.experimental import pallas as pl
from jax.experimental.pallas import tpu as pltpu
from jax.experimental.pallas import tpu_sc as plsc

NY = 496
NX = 432
NCH = 64
BATCH = 2
NPTS = 24000
NYNX = NY * NX
SLAB = NCH * NYNX
OUT_ELEMS = BATCH * SLAB

N_PER_BATCH = NPTS // BATCH
GROUPS_PER_BATCH = N_PER_BATCH // 16
TILE_GROUPS = -(-GROUPS_PER_BATCH // 16)
TILE_PTS = TILE_GROUPS * 16
IDX_ROWS = TILE_PTS * NCH // 128

ZCHUNK = 13392
ZDMAS = SLAB // 16 // ZCHUNK
TILE_ZELEMS = SLAB // 16

NCORES = 2
NSUB = 16


def _tile_starts():
    starts = []
    for c in range(NCORES):
        for s in range(NSUB):
            g0 = (s * GROUPS_PER_BATCH) // NSUB
            starts.append(c * N_PER_BATCH + g0 * 16)
    return starts


def _sc_scatter(val_windows, base):
    mesh = plsc.VectorSubcoreMesh(core_axis_name="c", subcore_axis_name="s",
                                  num_cores=NCORES, num_subcores=NSUB)

    @functools.partial(
        pl.kernel,
        out_type=jax.ShapeDtypeStruct((OUT_ELEMS,), jnp.float32),
        mesh=mesh,
        scratch_types=[
            pltpu.VMEM((ZCHUNK,), jnp.float32),
            pltpu.VMEM((TILE_PTS,), jnp.int32),
            pltpu.VMEM((TILE_PTS * NCH,), jnp.float32),
            pltpu.VMEM((TILE_PTS * NCH,), jnp.int32),
            pltpu.SemaphoreType.DMA,
            pltpu.SemaphoreType.DMA,
        ],
    )
    def body(win_hbm, base_hbm, out_hbm, zero_v, base_v, vals_v, idx_v, zsem, ssem):
        c = lax.axis_index("c")
        s = lax.axis_index("s")
        w = c * NSUB + s

        def zinit(i, _):
            zero_v[pl.ds(i * 16, 16)] = jnp.zeros((16,), jnp.float32)
            return _
        lax.fori_loop(0, ZCHUNK // 16, zinit, None)

        zbase = c * SLAB + s * TILE_ZELEMS

        def zfire(i):
            return pltpu.async_copy(
                zero_v, out_hbm.at[pl.ds(zbase + i * ZCHUNK, ZCHUNK)], zsem)

        def zwave(q, _):
            h = [zfire(q * 8 + t) for t in range(8)]
            for t in range(8):
                h[t].wait()
            return _
        lax.fori_loop(0, ZDMAS // 8, zwave, None)

        plsc.subcore_barrier()

        g0 = (s * GROUPS_PER_BATCH) // NSUB
        row0 = c * N_PER_BATCH + g0 * 16

        pltpu.sync_copy(base_hbm.at[pl.ds(row0, TILE_PTS)], base_v)
        pltpu.sync_copy(win_hbm.at[w], vals_v)

        def group(pg, _):
            bvec = base_v[pl.ds(pg * 16, 16)]
            for ch in range(NCH):
                flat = ch * TILE_PTS + pg * 16
                idx_v[pl.ds(flat, 16)] = bvec + ch * NYNX
            return _
        lax.fori_loop(0, TILE_GROUPS, group, None)

        pltpu.async_copy(vals_v, out_hbm.at[idx_v], ssem).wait()

    return body(val_windows, base)


def kernel(voxel_features, coords):
    coords = coords.astype(jnp.int32)
    base = (coords[:, 0] * NCH) * NYNX + coords[:, 2] * NX + coords[:, 3]
    vf_t = voxel_features.T
    wins = jnp.stack([lax.slice_in_dim(vf_t, st, st + TILE_PTS, axis=1)
                      for st in _tile_starts()])
    wins = wins.reshape(NCORES * NSUB, TILE_PTS * NCH)
    out_flat = _sc_scatter(wins, base)
    return out_flat.reshape(BATCH, NCH, NY, NX)


# speedup vs baseline: 2.5013x; 2.5013x over previous
import functools

import jax
import jax.numpy as jnp
from jax import lax
from jax.experimental import pallas as pl
from jax.experimental.pallas import tpu as pltpu
from jax.experimental.pallas import tpu_sc as plsc

NY = 496
NX = 432
NCH = 64
BATCH = 2
NPTS = 24000
N_PER_BATCH = NPTS // BATCH
NPB_PAD = 12032

NCORES = 2
NSUB = 16
CH_PER_TILE = NCH // NSUB

YB = 48
NBLK = 11
LAST_Y0 = NY - YB

SCAN_GROUPS = N_PER_BATCH // 16
SEL_CAP = N_PER_BATCH + 16
DUMP0 = SEL_CAP


def _sc_build(vf_t_flat, pos, yarr):
    mesh = plsc.VectorSubcoreMesh(core_axis_name="c", subcore_axis_name="s",
                                  num_cores=NCORES, num_subcores=NSUB)

    @functools.partial(
        pl.kernel,
        out_type=jax.ShapeDtypeStruct((BATCH * NCH * NY * NX,), jnp.float32),
        mesh=mesh,
        compiler_params=pltpu.CompilerParams(needs_layout_passes=False),
        scratch_types=[
            pltpu.VMEM((YB * NX + NX,), jnp.float32),
            pltpu.VMEM((YB * NX + NX,), jnp.float32),
            pltpu.VMEM((CH_PER_TILE, NPB_PAD), jnp.float32),
            pltpu.VMEM((NPB_PAD,), jnp.int32),
            pltpu.VMEM((NPB_PAD,), jnp.int32),
            pltpu.VMEM((SEL_CAP + 16,), jnp.int32),
            pltpu.SemaphoreType.DMA,
            pltpu.SemaphoreType.DMA,
        ],
    )
    def body(vf_hbm, pos_hbm, y_hbm, out_hbm,
             buf_a, buf_b, vf4, pos_v, y_v, sel_v, sem_a, sem_b):
        c = lax.axis_index("c")
        s = lax.axis_index("s")

        pltpu.sync_copy(pos_hbm.at[pl.ds(c * NPB_PAD, NPB_PAD)], pos_v)
        pltpu.sync_copy(y_hbm.at[pl.ds(c * NPB_PAD, NPB_PAD)], y_v)
        for chi in range(CH_PER_TILE):
            ch = s * CH_PER_TILE + chi
            pltpu.sync_copy(
                vf_hbm.at[pl.ds((ch * BATCH + c) * NPB_PAD, NPB_PAD)],
                vf4.at[chi])

        bufs = (buf_a, buf_b)
        sems = (sem_a, sem_b)
        iota = lax.iota(jnp.int32, 16)
        zeros16 = jnp.zeros((16,), jnp.float32)

        def binit(i, _):
            buf_a[pl.ds(i * 16, 16)] = zeros16
            buf_b[pl.ds(i * 16, 16)] = zeros16
            return _
        lax.fori_loop(0, (YB * NX + NX) // 16, binit, None)

        def wait_dma(which_buf, which_sem):
            pltpu.make_async_copy(
                which_buf.at[pl.ds(0, YB * NX)], out_hbm.at[pl.ds(0, YB * NX)],
                which_sem).wait()

        def blk_body(blk, _):
            y0 = pl.multiple_of(jnp.minimum(blk * YB, LAST_Y0), 8)

            def scan(i, cnt):
                yv = y_v[pl.ds(i * 16, 16)]
                m = (yv >= y0) & (yv < y0 + YB)
                inc = m.astype(jnp.int32)
                pref = plsc.cumsum(inc)
                dst = jnp.where(m, cnt + pref - 1, DUMP0 + iota)
                plsc.store_scatter(sel_v, [dst], iota + i * 16)
                return cnt + jnp.max(pref)
            cnt = lax.fori_loop(0, SCAN_GROUPS, scan, jnp.int32(0))
            sel_v[pl.ds(cnt, 16)] = jnp.zeros((16,), jnp.int32)
            n_grp = (cnt + 15) // 16

            def points(i):
                ids = sel_v[pl.ds(i * 16, 16)]
                valid = (iota + i * 16) < cnt
                yg = plsc.load_gather(y_v, [ids])
                pg = plsc.load_gather(pos_v, [ids])
                dst = jnp.where(valid, pg - y0 * NX, YB * NX)
                return ids, dst

            def build(chi, buf):
                ch_ids = jnp.full((16,), chi, jnp.int32)

                def one(i, _):
                    ids, dst = points(i)
                    vals = plsc.load_gather(vf4, [ch_ids, ids])
                    plsc.store_scatter(buf, [dst], vals)
                    return _
                lax.fori_loop(0, n_grp, one, None)

            def rezero(buf):
                def one(i, _):
                    _ids, dst = points(i)
                    plsc.store_scatter(buf, [dst], zeros16)
                    return _
                lax.fori_loop(0, n_grp, one, None)

            for chi in range(CH_PER_TILE):
                k = chi % 2
                if chi >= 2:
                    wait_dma(bufs[k], sems[k])
                    rezero(bufs[k])
                build(chi, bufs[k])
                ch = s * CH_PER_TILE + chi
                flatbase = ((c * NCH + ch) * NY + y0) * NX
                pltpu.async_copy(bufs[k].at[pl.ds(0, YB * NX)],
                                 out_hbm.at[pl.ds(flatbase, YB * NX)], sems[k])
            for k in range(2):
                wait_dma(bufs[k], sems[k])
                rezero(bufs[k])
            return _
        lax.fori_loop(0, NBLK, blk_body, None)

    return body(vf_t_flat, pos, yarr)


def kernel(voxel_features, coords):
    coords = coords.astype(jnp.int32)
    y = coords[:, 2]
    pos = y * NX + coords[:, 3]
    vf_seg = voxel_features.T.reshape(NCH, BATCH, N_PER_BATCH)
    vf_pad = jnp.pad(vf_seg, ((0, 0), (0, 0), (0, NPB_PAD - N_PER_BATCH)))
    pos_pad = jnp.pad(pos.reshape(BATCH, N_PER_BATCH),
                      ((0, 0), (0, NPB_PAD - N_PER_BATCH)))
    y_pad = jnp.pad(y.reshape(BATCH, N_PER_BATCH),
                    ((0, 0), (0, NPB_PAD - N_PER_BATCH)))
    out_flat = _sc_build(vf_pad.reshape(-1), pos_pad.reshape(-1),
                         y_pad.reshape(-1))
    return out_flat.reshape(BATCH, NCH, NY, NX)


# speedup vs baseline: 3.3067x; 1.3220x over previous
import functools

import jax
import jax.numpy as jnp
from jax import lax
from jax.experimental import pallas as pl
from jax.experimental.pallas import tpu as pltpu
from jax.experimental.pallas import tpu_sc as plsc

NY = 496
NX = 432
NCH = 64
BATCH = 2
NPTS = 24000
N_PER_BATCH = NPTS // BATCH
NPB_PAD = 12032

NCORES = 2
NSUB = 16
CH_PER_TILE = NCH // NSUB

YB = 48
NBLK = 11
LAST_Y0 = NY - YB

SCAN_GROUPS = N_PER_BATCH // 16
SEL_CAP = N_PER_BATCH + 16
DUMP0 = SEL_CAP


def _sc_build(vf_t_flat, pos, yarr):
    mesh = plsc.VectorSubcoreMesh(core_axis_name="c", subcore_axis_name="s",
                                  num_cores=NCORES, num_subcores=NSUB)

    @functools.partial(
        pl.kernel,
        out_type=jax.ShapeDtypeStruct((BATCH * NCH * NY * NX,), jnp.float32),
        mesh=mesh,
        compiler_params=pltpu.CompilerParams(needs_layout_passes=False),
        scratch_types=[
            pltpu.VMEM((YB * NX + NX,), jnp.float32),
            pltpu.VMEM((YB * NX + NX,), jnp.float32),
            pltpu.VMEM((CH_PER_TILE, NPB_PAD), jnp.float32),
            pltpu.VMEM((NPB_PAD,), jnp.int32),
            pltpu.VMEM((NPB_PAD,), jnp.int32),
            pltpu.VMEM((SEL_CAP + 16,), jnp.int32),
            pltpu.SemaphoreType.DMA,
            pltpu.SemaphoreType.DMA,
        ],
    )
    def body(vf_hbm, pos_hbm, y_hbm, out_hbm,
             buf_a, buf_b, vf4, pos_v, y_v, sel_v, sem_a, sem_b):
        c = lax.axis_index("c")
        s = lax.axis_index("s")

        pltpu.sync_copy(pos_hbm.at[pl.ds(c * NPB_PAD, NPB_PAD)], pos_v)
        pltpu.sync_copy(y_hbm.at[pl.ds(c * NPB_PAD, NPB_PAD)], y_v)
        for chi in range(CH_PER_TILE):
            ch = s * CH_PER_TILE + chi
            pltpu.sync_copy(
                vf_hbm.at[pl.ds((ch * BATCH + c) * NPB_PAD, NPB_PAD)],
                vf4.at[chi])

        bufs = (buf_a, buf_b)
        sems = (sem_a, sem_b)
        iota = lax.iota(jnp.int32, 16)
        zeros16 = jnp.zeros((16,), jnp.float32)

        def binit(i, _):
            buf_a[pl.ds(i * 16, 16)] = zeros16
            buf_b[pl.ds(i * 16, 16)] = zeros16
            return _
        lax.fori_loop(0, (YB * NX + NX) // 16, binit, None)

        def wait_dma(which_buf, which_sem):
            pltpu.make_async_copy(
                which_buf.at[pl.ds(0, YB * NX)], out_hbm.at[pl.ds(0, YB * NX)],
                which_sem).wait()

        def blk_body(blk, _):
            y0 = pl.multiple_of(jnp.minimum(blk * YB, LAST_Y0), 8)

            def scan(i, cnt):
                yv = y_v[pl.ds(i * 16, 16)]
                m = (yv >= y0) & (yv < y0 + YB)
                inc = m.astype(jnp.int32)
                pref = plsc.cumsum(inc)
                dst = jnp.where(m, cnt + pref - 1, DUMP0 + iota)
                plsc.store_scatter(sel_v, [dst], iota + i * 16)
                return cnt + jnp.max(pref)
            cnt = lax.fori_loop(0, SCAN_GROUPS, scan, jnp.int32(0))
            sel_v[pl.ds(cnt, 16)] = jnp.zeros((16,), jnp.int32)
            n_grp = (cnt + 15) // 16

            def points(i):
                ids = sel_v[pl.ds(i * 16, 16)]
                valid = (iota + i * 16) < cnt
                yg = plsc.load_gather(y_v, [ids])
                pg = plsc.load_gather(pos_v, [ids])
                dst = jnp.where(valid, pg - y0 * NX, YB * NX)
                return ids, dst

            def build(chi, buf):
                ch_ids = jnp.full((16,), chi, jnp.int32)

                def one(i, _):
                    ids, dst = points(i)
                    vals = plsc.load_gather(vf4, [ch_ids, ids])
                    plsc.store_scatter(buf, [dst], vals)
                    return _
                lax.fori_loop(0, n_grp, one, None)

            def rezero(buf):
                def one(i, _):
                    _ids, dst = points(i)
                    plsc.store_scatter(buf, [dst], zeros16)
                    return _
                lax.fori_loop(0, n_grp, one, None)

            for chi in range(CH_PER_TILE):
                k = chi % 2
                if chi >= 2:
                    wait_dma(bufs[k], sems[k])
                    rezero(bufs[k])
                build(chi, bufs[k])
                ch = s * CH_PER_TILE + chi
                flatbase = ((c * NCH + ch) * NY + y0) * NX
                pltpu.async_copy(bufs[k].at[pl.ds(0, YB * NX)],
                                 out_hbm.at[pl.ds(flatbase, YB * NX)], sems[k])
            for k in range(2):
                wait_dma(bufs[k], sems[k])
                rezero(bufs[k])
            return _
        lax.fori_loop(0, NBLK, blk_body, None)

    return body(vf_t_flat, pos, yarr)


def _sc_relayout(img):
    mesh = plsc.VectorSubcoreMesh(core_axis_name="c", subcore_axis_name="s",
                                  num_cores=NCORES, num_subcores=NSUB)

    @functools.partial(
        pl.kernel,
        out_type=jax.ShapeDtypeStruct((BATCH, NCH, NY, NX), jnp.float32),
        mesh=mesh,
        scratch_types=[
            pltpu.VMEM((YB * NX,), jnp.float32),
            pltpu.VMEM((YB, NX), jnp.float32),
            pltpu.VMEM((YB, NX), jnp.float32),
            pltpu.SemaphoreType.DMA,
            pltpu.SemaphoreType.DMA,
            pltpu.SemaphoreType.DMA,
        ],
    )
    def body(img_hbm, out_hbm, buf_flat, buf_a, buf_b, sem_f, sem_a, sem_b):
        c = lax.axis_index("c")
        s = lax.axis_index("s")
        bufs = (buf_a, buf_b)
        sems = (sem_a, sem_b)

        def wait_dma(which_buf, which_sem):
            pltpu.make_async_copy(
                which_buf, out_hbm.at[0, 0, pl.ds(0, YB), :], which_sem).wait()

        def blk_body(blk, _):
            y0 = pl.multiple_of(jnp.minimum(blk * YB, LAST_Y0), 8)
            for chi in range(CH_PER_TILE):
                k = chi % 2
                ch = s * CH_PER_TILE + chi
                flatbase = ((c * NCH + ch) * NY + y0) * NX
                pltpu.sync_copy(img_hbm.at[pl.ds(flatbase, YB * NX)], buf_flat)
                if chi >= 2:
                    wait_dma(bufs[k], sems[k])

                def rows(r, _r):
                    for g in range(NX // 16):
                        bufs[k][r, pl.ds(g * 16, 16)] = (
                            buf_flat[pl.ds(r * NX + g * 16, 16)])
                    return _r
                lax.fori_loop(0, YB, rows, None)
                pltpu.async_copy(bufs[k], out_hbm.at[c, ch, pl.ds(y0, YB), :],
                                 sems[k])
            for k in range(2):
                wait_dma(bufs[k], sems[k])
            return _
        lax.fori_loop(0, NBLK, blk_body, None)

    return body(img)


def kernel(voxel_features, coords):
    coords = coords.astype(jnp.int32)
    y = coords[:, 2]
    pos = y * NX + coords[:, 3]
    vf_seg = voxel_features.T.reshape(NCH, BATCH, N_PER_BATCH)
    vf_pad = jnp.pad(vf_seg, ((0, 0), (0, 0), (0, NPB_PAD - N_PER_BATCH)))
    pos_pad = jnp.pad(pos.reshape(BATCH, N_PER_BATCH),
                      ((0, 0), (0, NPB_PAD - N_PER_BATCH)))
    y_pad = jnp.pad(y.reshape(BATCH, N_PER_BATCH),
                    ((0, 0), (0, NPB_PAD - N_PER_BATCH)))
    img = _sc_build(vf_pad.reshape(-1), pos_pad.reshape(-1),
                    y_pad.reshape(-1))
    return _sc_relayout(img)


# speedup vs baseline: 3.5714x; 1.0801x over previous
import functools

import jax
import jax.numpy as jnp
from jax import lax
from jax.experimental import pallas as pl
from jax.experimental.pallas import tpu as pltpu
from jax.experimental.pallas import tpu_sc as plsc

NY = 496
NX = 432
NCH = 64
BATCH = 2
NPTS = 24000
N_PER_BATCH = NPTS // BATCH
NPB_PAD = 12032

NCORES = 2
NSUB = 16
CH_PER_TILE = NCH // NSUB

YB = 48
NBLK = 11
LAST_Y0 = NY - YB

SCAN_GROUPS = N_PER_BATCH // 16
SEL_CAP = N_PER_BATCH + 16
DUMP0 = SEL_CAP


def _sc_build(vf_t_flat, pos, yarr):
    mesh = plsc.VectorSubcoreMesh(core_axis_name="c", subcore_axis_name="s",
                                  num_cores=NCORES, num_subcores=NSUB)

    @functools.partial(
        pl.kernel,
        out_type=jax.ShapeDtypeStruct((BATCH * NCH * NY * NX,), jnp.float32),
        mesh=mesh,
        compiler_params=pltpu.CompilerParams(needs_layout_passes=False),
        scratch_types=[
            pltpu.VMEM((YB * NX + NX,), jnp.float32),
            pltpu.VMEM((YB * NX + NX,), jnp.float32),
            pltpu.VMEM((CH_PER_TILE, NPB_PAD), jnp.float32),
            pltpu.VMEM((NPB_PAD,), jnp.int32),
            pltpu.VMEM((NPB_PAD,), jnp.int32),
            pltpu.VMEM((SEL_CAP + 16,), jnp.int32),
            pltpu.SemaphoreType.DMA,
            pltpu.SemaphoreType.DMA,
        ],
    )
    def body(vf_hbm, pos_hbm, y_hbm, out_hbm,
             buf_a, buf_b, vf4, pos_v, y_v, sel_v, sem_a, sem_b):
        c = lax.axis_index("c")
        s = lax.axis_index("s")

        pltpu.sync_copy(pos_hbm.at[pl.ds(c * NPB_PAD, NPB_PAD)], pos_v)
        pltpu.sync_copy(y_hbm.at[pl.ds(c * NPB_PAD, NPB_PAD)], y_v)
        for chi in range(CH_PER_TILE):
            ch = s * CH_PER_TILE + chi
            pltpu.sync_copy(
                vf_hbm.at[pl.ds((ch * BATCH + c) * NPB_PAD, NPB_PAD)],
                vf4.at[chi])

        bufs = (buf_a, buf_b)
        sems = (sem_a, sem_b)
        iota = lax.iota(jnp.int32, 16)
        zeros16 = jnp.zeros((16,), jnp.float32)

        def binit(i, _):
            buf_a[pl.ds(i * 16, 16)] = zeros16
            buf_b[pl.ds(i * 16, 16)] = zeros16
            return _
        lax.fori_loop(0, (YB * NX + NX) // 16, binit, None)

        def wait_dma(which_buf, which_sem):
            pltpu.make_async_copy(
                which_buf.at[pl.ds(0, YB * NX)], out_hbm.at[pl.ds(0, YB * NX)],
                which_sem).wait()

        def blk_body(blk, _):
            y0 = pl.multiple_of(jnp.minimum(blk * YB, LAST_Y0), 8)

            def scan(i, cnt):
                yv = y_v[pl.ds(i * 16, 16)]
                m = (yv >= y0) & (yv < y0 + YB)
                inc = m.astype(jnp.int32)
                pref = plsc.cumsum(inc)
                dst = jnp.where(m, cnt + pref - 1, DUMP0 + iota)
                plsc.store_scatter(sel_v, [dst], iota + i * 16)
                return cnt + jnp.max(pref)
            cnt = lax.fori_loop(0, SCAN_GROUPS, scan, jnp.int32(0))
            sel_v[pl.ds(cnt, 16)] = jnp.zeros((16,), jnp.int32)
            n_grp = (cnt + 15) // 16

            def points(i):
                ids = sel_v[pl.ds(i * 16, 16)]
                valid = (iota + i * 16) < cnt
                yg = plsc.load_gather(y_v, [ids])
                pg = plsc.load_gather(pos_v, [ids])
                dst = jnp.where(valid, pg - y0 * NX, YB * NX)
                return ids, dst

            def build(chi, buf):
                ch_ids = jnp.full((16,), chi, jnp.int32)

                def one(i, _):
                    ids, dst = points(i)
                    vals = plsc.load_gather(vf4, [ch_ids, ids])
                    plsc.store_scatter(buf, [dst], vals)
                    return _
                lax.fori_loop(0, n_grp, one, None)

            def rezero(buf):
                def one(i, _):
                    _ids, dst = points(i)
                    plsc.store_scatter(buf, [dst], zeros16)
                    return _
                lax.fori_loop(0, n_grp, one, None)

            for chi in range(CH_PER_TILE):
                k = chi % 2
                if chi >= 2:
                    wait_dma(bufs[k], sems[k])
                    rezero(bufs[k])
                build(chi, bufs[k])
                ch = s * CH_PER_TILE + chi
                flatbase = ((c * NCH + ch) * NY + y0) * NX
                pltpu.async_copy(bufs[k].at[pl.ds(0, YB * NX)],
                                 out_hbm.at[pl.ds(flatbase, YB * NX)], sems[k])
            for k in range(2):
                wait_dma(bufs[k], sems[k])
                rezero(bufs[k])
            return _
        lax.fori_loop(0, NBLK, blk_body, None)

    return body(vf_t_flat, pos, yarr)


def _sc_relayout(img):
    mesh = plsc.VectorSubcoreMesh(core_axis_name="c", subcore_axis_name="s",
                                  num_cores=NCORES, num_subcores=NSUB)

    @functools.partial(
        pl.kernel,
        out_type=jax.ShapeDtypeStruct((BATCH, NCH, NY, NX), jnp.float32),
        mesh=mesh,
        scratch_types=[
            pltpu.VMEM((2, YB * NX), jnp.float32),
            pltpu.VMEM((YB, NX), jnp.float32),
            pltpu.VMEM((YB, NX), jnp.float32),
            pltpu.SemaphoreType.DMA,
            pltpu.SemaphoreType.DMA,
            pltpu.SemaphoreType.DMA,
            pltpu.SemaphoreType.DMA,
        ],
    )
    def body(img_hbm, out_hbm, buf_flat, buf_a, buf_b, sem_f, sem_g,
             sem_a, sem_b):
        c = lax.axis_index("c")
        s = lax.axis_index("s")
        bufs = (buf_a, buf_b)
        sems = (sem_a, sem_b)

        def ydst(blk, chi):
            y0 = pl.multiple_of(jnp.minimum(blk * YB, LAST_Y0), 8)
            ch = s * CH_PER_TILE + chi
            return y0, ch

        fsems = (sem_f, sem_g)

        def fire_load(blk, chi, slot):
            y0, ch = ydst(blk, chi)
            flatbase = ((c * NCH + ch) * NY + y0) * NX
            return pltpu.async_copy(img_hbm.at[pl.ds(flatbase, YB * NX)],
                                    buf_flat.at[slot], fsems[slot])

        def wait_load(slot):
            pltpu.make_async_copy(img_hbm.at[pl.ds(0, YB * NX)],
                                  buf_flat.at[slot], fsems[slot]).wait()

        def wait_out(which_buf, which_sem):
            pltpu.make_async_copy(
                which_buf, out_hbm.at[0, 0, pl.ds(0, YB), :], which_sem).wait()

        fire_load(0, 0, 0)

        def blk_body(blk, _):
            for chi in range(CH_PER_TILE):
                k = chi % 2
                slot = chi % 2
                y0, ch = ydst(blk, chi)
                if chi < CH_PER_TILE - 1:
                    fire_load(blk, chi + 1, 1 - slot)
                else:
                    pl.when(blk + 1 < NBLK)(
                        lambda: fire_load(blk + 1, 0, 1 - slot) and None)
                wait_load(slot)
                ireg = blk * CH_PER_TILE + chi
                pl.when(ireg >= 2)(lambda: wait_out(bufs[k], sems[k]))

                def rows(r, _r):
                    for g in range(NX // 16):
                        bufs[k][r, pl.ds(g * 16, 16)] = (
                            buf_flat[slot, pl.ds(r * NX + g * 16, 16)])
                    return _r
                lax.fori_loop(0, YB, rows, None)
                pltpu.async_copy(bufs[k], out_hbm.at[c, ch, pl.ds(y0, YB), :],
                                 sems[k])
            return _
        lax.fori_loop(0, NBLK, blk_body, None)
        for k in range(2):
            wait_out(bufs[k], sems[k])

    return body(img)


def kernel(voxel_features, coords):
    coords = coords.astype(jnp.int32)
    y = coords[:, 2]
    pos = y * NX + coords[:, 3]
    vf_seg = voxel_features.T.reshape(NCH, BATCH, N_PER_BATCH)
    vf_pad = jnp.pad(vf_seg, ((0, 0), (0, 0), (0, NPB_PAD - N_PER_BATCH)))
    pos_pad = jnp.pad(pos.reshape(BATCH, N_PER_BATCH),
                      ((0, 0), (0, NPB_PAD - N_PER_BATCH)))
    y_pad = jnp.pad(y.reshape(BATCH, N_PER_BATCH),
                    ((0, 0), (0, NPB_PAD - N_PER_BATCH)))
    img = _sc_build(vf_pad.reshape(-1), pos_pad.reshape(-1),
                    y_pad.reshape(-1))
    return _sc_relayout(img)
